# baseline (speedup 1.0000x reference)
import functools

import jax
import jax.numpy as jnp
from jax import lax
from jax.experimental import pallas as pl
from jax.experimental.pallas import tpu as pltpu
from jax.experimental.pallas import tpu_sc as plsc

EMB_DIM = 768
NUM_CORES = 2
NUM_SUBCORES = 16
NUM_WORKERS = NUM_CORES * NUM_SUBCORES
CHUNK = 32


def kernel(embeddings, input_ids):
    batch, hist = input_ids.shape
    n = batch * hist
    assert n % (8 * NUM_WORKERS) == 0
    per_worker = n // NUM_WORKERS
    assert per_worker % (2 * CHUNK) == 0
    n_chunks = per_worker // CHUNK

    ids = input_ids.reshape(n).astype(jnp.int32)
    mesh = plsc.VectorSubcoreMesh(core_axis_name="c", subcore_axis_name="s")

    @functools.partial(
        pl.kernel,
        mesh=mesh,
        out_type=jax.ShapeDtypeStruct((n, EMB_DIM), embeddings.dtype),
        scratch_types=[
            pltpu.VMEM((per_worker,), jnp.int32),
            pltpu.VMEM((2, CHUNK, EMB_DIM), jnp.float32),
            pltpu.VMEM_SHARED((NUM_SUBCORES, 2, CHUNK, EMB_DIM), jnp.float32),
            pltpu.SemaphoreType.DMA,
            pltpu.SemaphoreType.DMA,
            pltpu.SemaphoreType.DMA,
            pltpu.SemaphoreType.DMA,
        ],
    )
    def gather_kernel(table_hbm, idx_hbm, out_hbm, idx_v, rows_v, spm_out,
                      sg0, sg1, so0, so1):
        sid = lax.axis_index("s")
        wid = sid * NUM_CORES + lax.axis_index("c")
        base = wid * per_worker
        pltpu.sync_copy(idx_hbm.at[pl.ds(base, per_worker)], idx_v)

        sg = (sg0, sg1)
        so = (so0, so1)

        def gather_copy(cc, b):
            return pltpu.make_async_copy(
                table_hbm.at[idx_v.at[pl.ds(cc * CHUNK, CHUNK)]],
                rows_v.at[b], sg[b])

        def out_copy(cc, b):
            return pltpu.make_async_copy(
                spm_out.at[sid].at[b],
                out_hbm.at[pl.ds(base + cc * CHUNK, CHUNK)], so[b])

        for b in range(2):
            gather_copy(b, b).start()
        for b in range(2):
            gather_copy(b, b).wait()
            pltpu.sync_copy(rows_v.at[b], spm_out.at[sid].at[b])
            gather_copy(b + 2, b).start()
            out_copy(b, b).start()

        @pl.loop(2, n_chunks - 2, step=2)
        def _(c):
            for b in range(2):
                cc = c + b
                out_copy(cc - 2, b).wait()
                gather_copy(cc, b).wait()
                pltpu.sync_copy(rows_v.at[b], spm_out.at[sid].at[b])
                gather_copy(cc + 2, b).start()
                out_copy(cc, b).start()

        for b in range(2):
            cc = n_chunks - 2 + b
            out_copy(cc - 2, b).wait()
            gather_copy(cc, b).wait()
            pltpu.sync_copy(rows_v.at[b], spm_out.at[sid].at[b])
            out_copy(cc, b).start()
        for b in range(2):
            out_copy(n_chunks - 2 + b, b).wait()

    out = gather_kernel(embeddings, ids)
    return out.reshape(batch, hist, EMB_DIM)


# speedup vs baseline: 1.0020x; 1.0020x over previous
import functools

import jax
import jax.numpy as jnp
from jax import lax
from jax.experimental import pallas as pl
from jax.experimental.pallas import tpu as pltpu
from jax.experimental.pallas import tpu_sc as plsc

EMB_DIM = 768
NUM_CORES = 2
NUM_SUBCORES = 16
NUM_WORKERS = NUM_CORES * NUM_SUBCORES
CHUNK = 16
NLANES = 4


def kernel(embeddings, input_ids):
    batch, hist = input_ids.shape
    n = batch * hist
    assert n % (8 * NUM_WORKERS) == 0
    per_worker = n // NUM_WORKERS
    assert per_worker % (NLANES * CHUNK) == 0
    n_chunks = per_worker // CHUNK

    ids = input_ids.reshape(n).astype(jnp.int32)
    mesh = plsc.VectorSubcoreMesh(core_axis_name="c", subcore_axis_name="s")

    @functools.partial(
        pl.kernel,
        mesh=mesh,
        out_type=jax.ShapeDtypeStruct((n, EMB_DIM), embeddings.dtype),
        scratch_types=[
            pltpu.VMEM((per_worker,), jnp.int32),
            pltpu.VMEM((NLANES, CHUNK, EMB_DIM), jnp.float32),
            pltpu.VMEM_SHARED((NUM_SUBCORES, NLANES - 1, CHUNK, EMB_DIM),
                              jnp.float32),
        ] + [pltpu.SemaphoreType.DMA] * (2 * NLANES),
    )
    def gather_kernel(table_hbm, idx_hbm, out_hbm, idx_v, rows_v, spm_out,
                      *sems):
        sid = lax.axis_index("s")
        wid = sid * NUM_CORES + lax.axis_index("c")
        base = wid * per_worker
        pltpu.sync_copy(idx_hbm.at[pl.ds(base, per_worker)], idx_v)

        sg = sems[:NLANES]
        so = sems[NLANES:]

        def gather_copy(cc, b):
            return pltpu.make_async_copy(
                table_hbm.at[idx_v.at[pl.ds(cc * CHUNK, CHUNK)]],
                rows_v.at[b], sg[b])

        def direct_out(cc):
            return pltpu.make_async_copy(
                rows_v.at[0],
                out_hbm.at[pl.ds(base + cc * CHUNK, CHUNK)], so[0])

        def spm_slot(b):
            return spm_out.at[sid].at[b - 1]

        def bounce_out(cc, b):
            return pltpu.make_async_copy(
                spm_slot(b),
                out_hbm.at[pl.ds(base + cc * CHUNK, CHUNK)], so[b])

        def visit(cc, b, start_next, wait_prev):
            gather_copy(cc, b).wait()
            if b == 0:
                direct_out(cc).start()
                direct_out(cc).wait()
            else:
                if wait_prev:
                    bounce_out(cc - NLANES, b).wait()
                pltpu.sync_copy(rows_v.at[b], spm_slot(b))
                bounce_out(cc, b).start()
            if start_next:
                gather_copy(cc + NLANES, b).start()

        for b in range(NLANES):
            gather_copy(b, b).start()
        for b in range(NLANES):
            visit(b, b, True, False)

        @pl.loop(NLANES, n_chunks - NLANES, step=NLANES)
        def _(c):
            for b in range(NLANES):
                visit(c + b, b, True, True)

        for b in range(NLANES):
            visit(n_chunks - NLANES + b, b, False, True)
        for b in range(1, NLANES):
            bounce_out(n_chunks - NLANES + b, b).wait()

    out = gather_kernel(embeddings, ids)
    return out.reshape(batch, hist, EMB_DIM)
